# baseline (speedup 1.0000x reference)
import functools

import jax
import jax.numpy as jnp
from jax import lax
from jax.experimental import pallas as pl
from jax.experimental.pallas import tpu as pltpu
from jax.experimental.pallas import tpu_sc as plsc

N_NODES = 50000
N_EDGES = 800000
EMB = 100
DP = 128
L = 16
NSC = 2
NTILES = 16
CHUNK = 12544
NCHUNK = 4
NPAD = CHUNK * NCHUNK
STRIPE = CHUNK // NTILES
EB = 2880
NBLK = 18
ESTRIPE = EB * NBLK
EPAD = ESTRIPE * NTILES
G = 80
NWIN = EB // G
NPAIR = NWIN // 2
NSCALE = 7


def _sc_spmm_body(src_hbm, dst_hbm, w_hbm, xp_hbm, out_hbm,
                  srcv, dstv, wv,
                  csrc0, cldst0, cw0, rows0, sem0,
                  csrc1, cldst1, cw1, rows1, sem1, acc):
    cid = lax.axis_index("c")
    sid = lax.axis_index("s")

    zf = jnp.zeros((L,), jnp.float32)
    zi = jnp.zeros((L,), jnp.int32)

    def zrow0(i, _):
        for j in range(DP // L):
            rows0[i, pl.ds(j * L, L)] = zf
        return 0

    def zrow1(i, _):
        for j in range(DP // L):
            rows1[i, pl.ds(j * L, L)] = zf
        return 0

    def stage(t, csrc_b, cldst_b, cw_b, lo, hi):
        wbase = t * G
        for j in range(G // L):
            sl_in = pl.ds(wbase + j * L, L)
            dv = dstv[sl_in]
            m = (dv >= lo) & (dv < hi)
            sl_st = pl.ds(j * L, L)
            csrc_b[sl_st] = srcv[sl_in]
            cw_b[sl_st] = jnp.where(m, wv[sl_in], zf)
            cldst_b[sl_st] = jnp.where(m, dv - lo, zi)

    def fire(csrc_b, rows_b, sem_b):
        pltpu.async_copy(xp_hbm.at[csrc_b], rows_b, sem_b)

    def wait(csrc_b, rows_b, sem_b):
        pltpu.make_async_copy(xp_hbm.at[csrc_b], rows_b, sem_b).wait()

    def drain(cw_b, cldst_b, rows_b):
        def scale(g, _):
            g0 = 4 * g
            ws = []
            for r in range(4):
                wvec = cw_b[pl.ds(g0 + r, L)]
                ws.append(zf + wvec[0])
            for r in range(4):
                for j in range(NSCALE):
                    sl = pl.ds(j * L, L)
                    rows_b[g0 + r, sl] = rows_b[g0 + r, sl] * ws[r]
            return 0

        lax.fori_loop(0, G // 4, scale, 0)
        pltpu.sync_copy(rows_b, acc.at[cldst_b], add=True)

    for k in range(NCHUNK // NSC):
        chunk = cid * (NCHUNK // NSC) + k
        lo = chunk * CHUNK
        hi = lo + CHUNK
        base = sid * STRIPE

        lax.fori_loop(0, G, zrow0, 0)
        lax.fori_loop(0, G, zrow1, 0)
        for j in range(STRIPE // (2 * G)):
            pltpu.sync_copy(rows0, acc.at[pl.ds(base + 2 * j * G, G)])
            pltpu.sync_copy(rows1, acc.at[pl.ds(base + (2 * j + 1) * G, G)])
        tail = STRIPE % (2 * G)
        if tail:
            pltpu.sync_copy(rows0.at[pl.ds(0, tail)],
                            acc.at[pl.ds(base + STRIPE - tail, tail)])
        plsc.subcore_barrier()

        def block(b, _):
            ebase = sid * ESTRIPE + b * EB
            c1 = pltpu.async_copy(src_hbm.at[pl.ds(ebase, EB)], srcv, sem0)
            c2 = pltpu.async_copy(dst_hbm.at[pl.ds(ebase, EB)], dstv, sem1)
            c3 = pltpu.async_copy(w_hbm.at[pl.ds(ebase, EB)], wv, sem0)
            c1.wait()
            c2.wait()
            c3.wait()

            stage(0, csrc0, cldst0, cw0, lo, hi)
            fire(csrc0, rows0, sem0)

            def pair(p, _2):
                stage(2 * p + 1, csrc1, cldst1, cw1, lo, hi)
                fire(csrc1, rows1, sem1)
                wait(csrc0, rows0, sem0)
                drain(cw0, cldst0, rows0)

                @pl.when(p < NPAIR - 1)
                def _():
                    stage(2 * p + 2, csrc0, cldst0, cw0, lo, hi)
                    fire(csrc0, rows0, sem0)

                wait(csrc1, rows1, sem1)
                drain(cw1, cldst1, rows1)
                return 0

            lax.fori_loop(0, NPAIR, pair, 0)
            return 0

        lax.fori_loop(0, NBLK, block, 0)
        plsc.subcore_barrier()

        pltpu.sync_copy(acc.at[pl.ds(base, STRIPE)],
                        out_hbm.at[pl.ds(lo + base, STRIPE)])
        plsc.subcore_barrier()


@jax.jit
def _sc_spmm(src, dst, w, xp):
    mesh = plsc.VectorSubcoreMesh(core_axis_name="c", subcore_axis_name="s")
    f = functools.partial(
        pl.kernel,
        mesh=mesh,
        out_type=jax.ShapeDtypeStruct((NPAD, DP), jnp.float32),
        scratch_types=[
            pltpu.VMEM((EB,), jnp.int32),
            pltpu.VMEM((EB,), jnp.int32),
            pltpu.VMEM((EB,), jnp.float32),
            pltpu.VMEM((G,), jnp.int32),
            pltpu.VMEM((G,), jnp.int32),
            pltpu.VMEM((G + L,), jnp.float32),
            pltpu.VMEM((G, DP), jnp.float32),
            pltpu.SemaphoreType.DMA,
            pltpu.VMEM((G,), jnp.int32),
            pltpu.VMEM((G,), jnp.int32),
            pltpu.VMEM((G + L,), jnp.float32),
            pltpu.VMEM((G, DP), jnp.float32),
            pltpu.SemaphoreType.DMA,
            pltpu.VMEM_SHARED((CHUNK, DP), jnp.float32),
        ],
    )(_sc_spmm_body)
    return f(src, dst, w, xp)


def _tc_body(s_ref, wt_ref, o_ref):
    r = jnp.dot(s_ref[...], wt_ref[...], preferred_element_type=jnp.float32)
    nrm = jnp.sqrt(jnp.sum(r * r, axis=1, keepdims=True))
    o_ref[...] = r / jnp.maximum(nrm, 1e-12)


@jax.jit
def _tc_linear_normalize(s, wt):
    nblk = 16
    rb = NPAD // nblk
    return pl.pallas_call(
        _tc_body,
        grid=(nblk,),
        in_specs=[
            pl.BlockSpec((rb, DP), lambda i: (i, 0)),
            pl.BlockSpec((DP, DP), lambda i: (0, 0)),
        ],
        out_specs=pl.BlockSpec((rb, DP), lambda i: (i, 0)),
        out_shape=jax.ShapeDtypeStruct((NPAD, DP), jnp.float32),
    )(s, wt)


def kernel(x, edge_index, edge_weight, W):
    src = edge_index[0].astype(jnp.int32)
    dst = edge_index[1].astype(jnp.int32)
    epad = EPAD - N_EDGES
    src = jnp.pad(src, (0, epad))
    dst = jnp.pad(dst, (0, epad))
    w = jnp.pad(edge_weight, (0, epad))
    xp = jnp.pad(x, ((0, 0), (0, DP - EMB)))
    wt = jnp.pad(W.T, ((0, DP - EMB), (0, DP - EMB)))
    s = _sc_spmm(src, dst, w, xp)
    h = _tc_linear_normalize(s, wt)
    return h[:N_NODES, :EMB]


# speedup vs baseline: 2.5459x; 2.5459x over previous
import functools

import jax
import jax.numpy as jnp
from jax import lax
from jax.experimental import pallas as pl
from jax.experimental.pallas import tpu as pltpu
from jax.experimental.pallas import tpu_sc as plsc

N_NODES = 50000
N_EDGES = 800000
EMB = 100
DP = 128
L = 16
NSC = 2
NTILES = 16
CHUNK = 12544
NCHUNK = 4
NPAD = CHUNK * NCHUNK
STRIPE = CHUNK // NTILES
EB = 3584
NBLK = 14
ESTRIPE = EB * NBLK
EPAD = ESTRIPE * NTILES
G = 64
NWIN = EB // G
NPAIR = NWIN // 2
NSCALE = 7


def _sc_spmm_body(src_hbm, dst_hbm, w_hbm, xp_hbm, out_hbm,
                  srcv, dstv, wv,
                  csrc0, cldst0, cw0, rows0, sem0,
                  csrc1, cldst1, cw1, rows1, sem1, acc):
    cid = lax.axis_index("c")
    sid = lax.axis_index("s")

    zf = jnp.zeros((L,), jnp.float32)
    zi = jnp.zeros((L,), jnp.int32)

    def zrow0(i, _):
        for j in range(DP // L):
            rows0[i, pl.ds(j * L, L)] = zf
        return 0

    def zrow1(i, _):
        for j in range(DP // L):
            rows1[i, pl.ds(j * L, L)] = zf
        return 0

    def stage(t, csrc_b, cldst_b, cw_b, lo, hi):
        wbase = t * G
        for j in range(G // L):
            sl_in = pl.ds(wbase + j * L, L)
            dv = dstv[sl_in]
            m = (dv >= lo) & (dv < hi)
            sl_st = pl.ds(j * L, L)
            csrc_b[sl_st] = srcv[sl_in]
            cw_b[sl_st] = jnp.where(m, wv[sl_in], zf)
            cldst_b[sl_st] = jnp.where(m, dv - lo, zi)

    def fire(csrc_b, rows_b, sem_b):
        pltpu.async_copy(xp_hbm.at[csrc_b], rows_b, sem_b)

    def wait(csrc_b, rows_b, sem_b):
        pltpu.make_async_copy(xp_hbm.at[csrc_b], rows_b, sem_b).wait()

    def drain(cw_b, cldst_b, rows_b):
        def scale(g, _):
            g0 = 4 * g
            ws = []
            for r in range(4):
                wvec = cw_b[pl.ds(g0 + r, L)]
                ws.append(zf + wvec[0])
            for r in range(4):
                for j in range(NSCALE):
                    sl = pl.ds(j * L, L)
                    rows_b[g0 + r, sl] = rows_b[g0 + r, sl] * ws[r]
            return 0

        lax.fori_loop(0, G // 4, scale, 0)
        pltpu.sync_copy(rows_b, acc.at[cldst_b], add=True)

    for k in range(NCHUNK // NSC):
        chunk = cid * (NCHUNK // NSC) + k
        lo = chunk * CHUNK
        hi = lo + CHUNK
        base = sid * STRIPE

        lax.fori_loop(0, G, zrow0, 0)
        lax.fori_loop(0, G, zrow1, 0)
        for j in range(STRIPE // (2 * G)):
            pltpu.sync_copy(rows0, acc.at[pl.ds(base + 2 * j * G, G)])
            pltpu.sync_copy(rows1, acc.at[pl.ds(base + (2 * j + 1) * G, G)])
        tail = STRIPE % (2 * G)
        if tail:
            pltpu.sync_copy(rows0.at[pl.ds(0, tail)],
                            acc.at[pl.ds(base + STRIPE - tail, tail)])
        plsc.subcore_barrier()

        def block(b, _):
            ebase = sid * ESTRIPE + b * EB
            c1 = pltpu.async_copy(src_hbm.at[pl.ds(ebase, EB)], srcv, sem0)
            c2 = pltpu.async_copy(dst_hbm.at[pl.ds(ebase, EB)], dstv, sem1)
            c3 = pltpu.async_copy(w_hbm.at[pl.ds(ebase, EB)], wv, sem0)
            c1.wait()
            c2.wait()
            c3.wait()

            stage(0, csrc0, cldst0, cw0, lo, hi)
            fire(csrc0, rows0, sem0)

            def pair(p, _2):
                stage(2 * p + 1, csrc1, cldst1, cw1, lo, hi)
                fire(csrc1, rows1, sem1)
                wait(csrc0, rows0, sem0)
                drain(cw0, cldst0, rows0)

                @pl.when(p < NPAIR - 1)
                def _():
                    stage(2 * p + 2, csrc0, cldst0, cw0, lo, hi)
                    fire(csrc0, rows0, sem0)

                wait(csrc1, rows1, sem1)
                drain(cw1, cldst1, rows1)
                return 0

            lax.fori_loop(0, NPAIR, pair, 0)
            return 0

        lax.fori_loop(0, NBLK, block, 0)
        plsc.subcore_barrier()

        pltpu.sync_copy(acc.at[pl.ds(base, STRIPE)],
                        out_hbm.at[pl.ds(lo + base, STRIPE)])
        plsc.subcore_barrier()


@jax.jit
def _sc_spmm(src, dst, w, xp):
    mesh = plsc.VectorSubcoreMesh(core_axis_name="c", subcore_axis_name="s")
    f = functools.partial(
        pl.kernel,
        mesh=mesh,
        out_type=jax.ShapeDtypeStruct((NPAD, DP), jnp.float32),
        scratch_types=[
            pltpu.VMEM((EB,), jnp.int32),
            pltpu.VMEM((EB,), jnp.int32),
            pltpu.VMEM((EB,), jnp.float32),
            pltpu.VMEM((G,), jnp.int32),
            pltpu.VMEM((G,), jnp.int32),
            pltpu.VMEM((G + L,), jnp.float32),
            pltpu.VMEM((G, DP), jnp.float32),
            pltpu.SemaphoreType.DMA,
            pltpu.VMEM((G,), jnp.int32),
            pltpu.VMEM((G,), jnp.int32),
            pltpu.VMEM((G + L,), jnp.float32),
            pltpu.VMEM((G, DP), jnp.float32),
            pltpu.SemaphoreType.DMA,
            pltpu.VMEM_SHARED((CHUNK, DP), jnp.float32),
        ],
    )(_sc_spmm_body)
    return f(src, dst, w, xp)


def _tc_body(s_ref, wt_ref, o_ref):
    r = jnp.dot(s_ref[...], wt_ref[...], preferred_element_type=jnp.float32)
    nrm = jnp.sqrt(jnp.sum(r * r, axis=1, keepdims=True))
    o_ref[...] = r / jnp.maximum(nrm, 1e-12)


@jax.jit
def _tc_linear_normalize(s, wt):
    nblk = 16
    rb = NPAD // nblk
    return pl.pallas_call(
        _tc_body,
        grid=(nblk,),
        in_specs=[
            pl.BlockSpec((rb, DP), lambda i: (i, 0)),
            pl.BlockSpec((DP, DP), lambda i: (0, 0)),
        ],
        out_specs=pl.BlockSpec((rb, DP), lambda i: (i, 0)),
        out_shape=jax.ShapeDtypeStruct((NPAD, DP), jnp.float32),
    )(s, wt)


def kernel(x, edge_index, edge_weight, W):
    src = edge_index[0].astype(jnp.int32)
    dst = edge_index[1].astype(jnp.int32)
    epad = EPAD - N_EDGES
    src = jnp.pad(src, (0, epad))
    dst = jnp.pad(dst, (0, epad))
    w = jnp.pad(edge_weight, (0, epad))
    xp = jnp.pad(x, ((0, 0), (0, DP - EMB)))
    wt = jnp.pad(W.T, ((0, DP - EMB), (0, DP - EMB)))
    s = _sc_spmm(src, dst, w, xp)
    h = _tc_linear_normalize(s, wt)
    return h[:N_NODES, :EMB]
